# baseline (speedup 1.0000x reference)
import jax
import jax.numpy as jnp
from jax.experimental import pallas as pl
from jax.experimental.pallas import tpu as pltpu

_Q = 4


def _pick_bt(B, C, HW, itemsize, target_bytes):
    per_b = C * HW * itemsize
    cap = max(1, target_bytes // per_b)
    bt = 1
    for d in range(1, min(B, cap) + 1):
        if B % d == 0:
            bt = d
    return bt


def _make_outer(*, T, bt, Q, C, HW, inv_hw):
    Cq = C // Q

    def outer(x_hbm, w1t_ref, b1_ref, w2t_ref, b2_ref, o_hbm):
        def inner(*refs):
            x_refs, o_refs = refs[:Q], refs[Q:]
            xs = [r[...] for r in x_refs]
            pooled = jnp.concatenate(
                [jnp.sum(xq, axis=-1, dtype=jnp.float32) for xq in xs],
                axis=1) * inv_hw
            h = jnp.dot(pooled, w1t_ref[...],
                        preferred_element_type=jnp.float32) + b1_ref[...]
            h = jnp.maximum(h, 0.0)
            s = jnp.dot(h, w2t_ref[...],
                        preferred_element_type=jnp.float32) + b2_ref[...]
            s = jax.nn.sigmoid(s)
            for q in range(Q):
                o_refs[q][...] = xs[q] * s[:, q * Cq:(q + 1) * Cq, None]

        spec = [
            pl.BlockSpec((bt, Cq, HW), lambda i, q=q: (i, q, 0))
            for q in range(Q)
        ]
        pipe = pltpu.emit_pipeline(
            inner, grid=(T,), in_specs=spec, out_specs=spec)
        pipe(*([x_hbm] * Q), *([o_hbm] * Q))

    return outer


@jax.jit
def kernel(x, w1, b1, w2, b2):
    B, C, H, W = x.shape
    Cr = w1.shape[0]
    HW = H * W
    xf = x.reshape(B, C, HW)
    w1t = w1.reshape(Cr, C).T
    w2t = w2.reshape(C, Cr).T
    b1r = b1.reshape(1, Cr)
    b2r = b2.reshape(1, C)

    Q = _Q
    while C % Q != 0 and Q > 1:
        Q //= 2
    bt = _pick_bt(B, C, HW, xf.dtype.itemsize, 8 * 1024 * 1024)
    T = B // bt

    outer = _make_outer(T=T, bt=bt, Q=Q, C=C, HW=HW, inv_hw=1.0 / HW)

    out = pl.pallas_call(
        outer,
        out_shape=jax.ShapeDtypeStruct((B, C, HW), xf.dtype),
        in_specs=[
            pl.BlockSpec(memory_space=pl.ANY),
            pl.BlockSpec((C, Cr), lambda: (0, 0)),
            pl.BlockSpec((1, Cr), lambda: (0, 0)),
            pl.BlockSpec((Cr, C), lambda: (0, 0)),
            pl.BlockSpec((1, C), lambda: (0, 0)),
        ],
        out_specs=pl.BlockSpec(memory_space=pl.ANY),
        compiler_params=pltpu.CompilerParams(
            vmem_limit_bytes=56 * 1024 * 1024,
        ),
    )(xf, w1t, b1r, w2t, b2r)
    return out.reshape(B, C, H, W)
